# baseline (speedup 1.0000x reference)
import functools

import jax
import jax.numpy as jnp
from jax import lax
from jax.experimental import pallas as pl
from jax.experimental.pallas import tpu as pltpu
from jax.experimental.pallas import tpu_sc as plsc

_B = 16384
_D = 16
_CHUNK = 128


@functools.lru_cache(maxsize=None)
def _build(NC: int, NS: int):
    NW = NC * NS
    b_per_w = _B // NW
    n_chunks = b_per_w // _CHUNK
    mesh = plsc.VectorSubcoreMesh(core_axis_name="c", subcore_axis_name="s")

    @functools.partial(
        pl.kernel,
        mesh=mesh,
        compiler_params=pltpu.CompilerParams(use_tc_tiling_on_sc=False),
        out_type=(
            jax.ShapeDtypeStruct((NW, n_chunks, _CHUNK, _D), jnp.float32),
            jax.ShapeDtypeStruct((NW, n_chunks, _CHUNK, _D), jnp.float32),
        ),
        scratch_types=[
            pltpu.VMEM((n_chunks, _CHUNK), jnp.int32),
            pltpu.VMEM((n_chunks, _CHUNK), jnp.int32),
            pltpu.VMEM((n_chunks, _CHUNK, _D), jnp.float32),
            pltpu.VMEM((n_chunks, _CHUNK, _D), jnp.float32),
            pltpu.SemaphoreType.DMA,
            pltpu.SemaphoreType.DMA,
        ],
    )
    def k(idx_hbm, pidx_hbm, W_hbm, Wp_hbm, out_hbm, pout_hbm,
          idx_v, pidx_v, rows_v, prows_v, sem_a, sem_b):
        wid = lax.axis_index("s") * NC + lax.axis_index("c")
        pltpu.sync_copy(idx_hbm.at[wid], idx_v)
        pltpu.sync_copy(pidx_hbm.at[wid], pidx_v)
        copies = []
        for j in range(n_chunks):
            copies.append(pltpu.async_copy(W_hbm.at[idx_v.at[j]], rows_v.at[j], sem_a))
            copies.append(pltpu.async_copy(Wp_hbm.at[pidx_v.at[j]], prows_v.at[j], sem_b))
        for c in copies:
            c.wait()
        pltpu.sync_copy(rows_v, out_hbm.at[wid])
        pltpu.sync_copy(prows_v, pout_hbm.at[wid])

    return k


def kernel(batch, persona_batch, W, W_persona):
    info = plsc.get_sparse_core_info()
    NC, NS = info.num_cores, info.num_subcores
    NW = NC * NS
    b_per_w = _B // NW
    n_chunks = b_per_w // _CHUNK
    idx = batch.astype(jnp.int32).reshape(NW, n_chunks, _CHUNK)
    pidx = persona_batch.astype(jnp.int32).reshape(NW, n_chunks, _CHUNK)
    out, pout = _build(NC, NS)(idx, pidx, W, W_persona)
    return out.reshape(_B, _D), pout.reshape(_B, _D)
